# Initial kernel scaffold; baseline (speedup 1.0000x reference)
import jax
import jax.numpy as jnp
from jax.experimental import pallas as pl


def kernel(log_probs, log_probs_lens, word_ids, target_lengths):
    raise NotImplementedError("write your pallas kernel here")


# speedup vs baseline: 35.2667x; 35.2667x over previous
import functools

import jax
import jax.numpy as jnp
from jax.experimental import pallas as pl
from jax.experimental.pallas import tpu as pltpu

B, T, V, U = 16, 2048, 512, 256
S = 2 * U + 1
S_PAD = 640
LSE_LANE = S + 1
NEG = -1e30
T_BLK = 128
NT = T // T_BLK


def _fwd_kernel(ext_ref, skip_ref, lens_ref, lp_ref, num_ref, den_ref,
                e_scratch, alpha_ref):
    pid = pl.program_id(0)

    @pl.when(pid == 0)
    def _init():
        lane = jax.lax.broadcasted_iota(jnp.int32, (B, S_PAD), 1)
        alpha_ref[...] = jnp.where(lane == 0, 0.0, NEG).astype(jnp.float32)
        num_ref[...] = jnp.zeros((B, 1), jnp.float32)
        den_ref[...] = jnp.zeros((B, 1), jnp.float32)

    iota_v = jax.lax.broadcasted_iota(jnp.int32, (V, S_PAD), 0)
    lane_s = jax.lax.broadcasted_iota(jnp.int32, (1, S_PAD), 1)
    for b in range(B):
        lp_b = lp_ref[b]
        onehot = (ext_ref[b:b + 1, :] == iota_v).astype(jnp.float32)
        e_b = jnp.dot(lp_b, onehot, preferred_element_type=jnp.float32)
        m = jnp.max(lp_b, axis=1, keepdims=True)
        lse = m + jnp.log(jnp.sum(jnp.exp(lp_b - m), axis=1, keepdims=True))
        e_b = e_b + jnp.where(lane_s == LSE_LANE, lse, 0.0)
        e_scratch[:, b, :] = e_b

    skip_neg = skip_ref[...]
    lens = lens_ref[...]
    lane = jax.lax.broadcasted_iota(jnp.int32, (B, S_PAD), 1)
    a2_neg = jnp.where(lane == 0, NEG, 0.0).astype(jnp.float32)

    def body(tt, carry):
        alpha, num, den = carry
        e = e_scratch[tt]
        a2 = pltpu.roll(alpha, 1, 1) + a2_neg
        a3 = pltpu.roll(alpha, 2, 1) + skip_neg
        m = jnp.maximum(jnp.maximum(alpha, a2), a3)
        new = m + jnp.log(jnp.exp(alpha - m) + jnp.exp(a2 - m)
                          + jnp.exp(a3 - m)) + e
        t = pid * T_BLK + tt
        sc = jnp.logaddexp(new[:, S - 2:S - 1], new[:, S - 1:S])
        num = jnp.where(lens == t + 1, sc, num)
        den = den + jnp.where(lens > t, e[:, LSE_LANE:LSE_LANE + 1], 0.0)
        return new, num, den

    alpha, num, den = jax.lax.fori_loop(
        0, T_BLK, body, (alpha_ref[...], num_ref[...], den_ref[...]))
    alpha_ref[...] = alpha
    num_ref[...] = num
    den_ref[...] = den


@jax.jit
def _graph_loss_impl(log_probs, log_probs_lens, word_ids, target_lengths):
    tgt = word_ids.astype(jnp.int32)
    ext = jnp.zeros((B, S), dtype=jnp.int32).at[:, 1::2].set(tgt)
    ext = jnp.concatenate(
        [ext, jnp.full((B, S_PAD - S), -1, jnp.int32)], axis=1)
    allow = jnp.concatenate(
        [jnp.zeros((B, 2), bool),
         (ext[:, 2:S] != 0) & (ext[:, 2:S] != ext[:, :S - 2])], axis=1)
    allow = jnp.concatenate(
        [allow, jnp.zeros((B, S_PAD - S), bool)], axis=1)
    skip_neg = jnp.where(allow, 0.0, NEG).astype(jnp.float32)
    lens = log_probs_lens.astype(jnp.int32).reshape(B, 1)

    num, den = pl.pallas_call(
        _fwd_kernel,
        grid=(NT,),
        in_specs=[
            pl.BlockSpec((B, S_PAD), lambda i: (0, 0)),
            pl.BlockSpec((B, S_PAD), lambda i: (0, 0)),
            pl.BlockSpec((B, 1), lambda i: (0, 0)),
            pl.BlockSpec((B, T_BLK, V), lambda i: (0, i, 0)),
        ],
        out_specs=[
            pl.BlockSpec((B, 1), lambda i: (0, 0)),
            pl.BlockSpec((B, 1), lambda i: (0, 0)),
        ],
        out_shape=[
            jax.ShapeDtypeStruct((B, 1), jnp.float32),
            jax.ShapeDtypeStruct((B, 1), jnp.float32),
        ],
        scratch_shapes=[
            pltpu.VMEM((T_BLK, B, S_PAD), jnp.float32),
            pltpu.VMEM((B, S_PAD), jnp.float32),
        ],
    )(ext, skip_neg, lens, log_probs)

    tl = target_lengths.astype(jnp.float32)
    num_loss = -num[:, 0]
    den_loss = -den[:, 0]
    return jnp.mean(num_loss / tl) - jnp.mean(den_loss / tl)


def kernel(log_probs, log_probs_lens, word_ids, target_lengths):
    return _graph_loss_impl(log_probs, log_probs_lens, word_ids,
                            target_lengths)


# speedup vs baseline: 63.4690x; 1.7997x over previous
import functools

import jax
import jax.numpy as jnp
from jax.experimental import pallas as pl
from jax.experimental.pallas import tpu as pltpu

B, T, V, U = 16, 2048, 512, 256
S = 2 * U + 1
S_PAD = 640
TAIL0 = 384
NEG = -1e30
T_BLK = 128
NT = T // T_BLK


def _fwd_kernel(ext_ref, skip_ref, lens_ref, lp_ref, num_ref, den_ref,
                e_scratch, alpha_ref):
    pid = pl.program_id(0)

    @pl.when(pid == 0)
    def _init():
        lane = jax.lax.broadcasted_iota(jnp.int32, (B, S_PAD), 1)
        alpha_ref[...] = jnp.where(lane == 0, 0.0, NEG).astype(jnp.float32)
        num_ref[...] = jnp.zeros((B, 1), jnp.float32)
        den_ref[...] = jnp.zeros((B, 1), jnp.float32)

    iota_v = jax.lax.broadcasted_iota(jnp.int32, (V, S_PAD), 0)
    row_t = (jax.lax.broadcasted_iota(jnp.int32, (T_BLK, 1), 0)
             + pid * T_BLK)
    for b in range(B):
        lp_b = lp_ref[b]
        onehot = (ext_ref[b:b + 1, :] == iota_v).astype(jnp.float32)
        e_scratch[:, b, :] = jnp.dot(lp_b, onehot,
                                     preferred_element_type=jnp.float32)
        m = jnp.max(lp_b, axis=1, keepdims=True)
        lse = m + jnp.log(jnp.sum(jnp.exp(lp_b - m), axis=1, keepdims=True))
        dpart = jnp.sum(jnp.where(row_t < lens_ref[b:b + 1, 0:1], lse, 0.0),
                        axis=0, keepdims=True)
        den_ref[b:b + 1, :] = den_ref[b:b + 1, :] + dpart

    skip_neg = skip_ref[...]
    lens = lens_ref[...]
    lane = jax.lax.broadcasted_iota(jnp.int32, (B, S_PAD), 1)
    a2_neg = jnp.where(lane == 0, NEG, 0.0).astype(jnp.float32)

    def body(tt, carry):
        alpha, tail = carry
        e = e_scratch[tt]
        a2 = pltpu.roll(alpha, 1, 1) + a2_neg
        a3 = pltpu.roll(alpha, 2, 1) + skip_neg
        m = jnp.maximum(jnp.maximum(alpha, a2), a3)
        new = m + jnp.log(jnp.exp(alpha - m) + jnp.exp(a2 - m)
                          + jnp.exp(a3 - m)) + e
        t = pid * T_BLK + tt
        tail = jnp.where(lens == t + 1, new[:, TAIL0:], tail)
        return new, tail

    tail0 = jnp.full((B, S_PAD - TAIL0), NEG, jnp.float32)
    alpha, tail = jax.lax.fori_loop(
        0, T_BLK, body, (alpha_ref[...], tail0))
    alpha_ref[...] = alpha
    sc = jnp.logaddexp(tail[:, S - 2 - TAIL0:S - 1 - TAIL0],
                       tail[:, S - 1 - TAIL0:S - TAIL0])
    hit = ((lens > pid * T_BLK) & (lens <= (pid + 1) * T_BLK))
    num_ref[...] = jnp.where(hit, sc, num_ref[...])


@jax.jit
def _graph_loss_impl(log_probs, log_probs_lens, word_ids, target_lengths):
    tgt = word_ids.astype(jnp.int32)
    ext = jnp.zeros((B, S), dtype=jnp.int32).at[:, 1::2].set(tgt)
    ext = jnp.concatenate(
        [ext, jnp.full((B, S_PAD - S), -1, jnp.int32)], axis=1)
    allow = jnp.concatenate(
        [jnp.zeros((B, 2), bool),
         (ext[:, 2:S] != 0) & (ext[:, 2:S] != ext[:, :S - 2])], axis=1)
    allow = jnp.concatenate(
        [allow, jnp.zeros((B, S_PAD - S), bool)], axis=1)
    skip_neg = jnp.where(allow, 0.0, NEG).astype(jnp.float32)
    lens = log_probs_lens.astype(jnp.int32).reshape(B, 1)

    num, den = pl.pallas_call(
        _fwd_kernel,
        grid=(NT,),
        in_specs=[
            pl.BlockSpec((B, S_PAD), lambda i: (0, 0)),
            pl.BlockSpec((B, S_PAD), lambda i: (0, 0)),
            pl.BlockSpec((B, 1), lambda i: (0, 0)),
            pl.BlockSpec((B, T_BLK, V), lambda i: (0, i, 0)),
        ],
        out_specs=[
            pl.BlockSpec((B, 1), lambda i: (0, 0)),
            pl.BlockSpec((B, 1), lambda i: (0, 0)),
        ],
        out_shape=[
            jax.ShapeDtypeStruct((B, 1), jnp.float32),
            jax.ShapeDtypeStruct((B, 1), jnp.float32),
        ],
        scratch_shapes=[
            pltpu.VMEM((T_BLK, B, S_PAD), jnp.float32),
            pltpu.VMEM((B, S_PAD), jnp.float32),
        ],
    )(ext, skip_neg, lens, log_probs)

    tl = target_lengths.astype(jnp.float32)
    num_loss = -num[:, 0]
    den_loss = -den[:, 0]
    return jnp.mean(num_loss / tl) - jnp.mean(den_loss / tl)


def kernel(log_probs, log_probs_lens, word_ids, target_lengths):
    return _graph_loss_impl(log_probs, log_probs_lens, word_ids,
                            target_lengths)
